# Initial kernel scaffold; baseline (speedup 1.0000x reference)
import jax
import jax.numpy as jnp
from jax.experimental import pallas as pl


def kernel(clauses, variable_count, clauses_count, params):
    raise NotImplementedError("write your pallas kernel here")


# speedup vs baseline: 4.9101x; 4.9101x over previous
import functools

import jax
import jax.numpy as jnp
from jax import lax
from jax.experimental import pallas as pl
from jax.experimental.pallas import tpu as pltpu
from jax.experimental.pallas import tpu_sc as plsc

NV = 10000
NC = 42000
NG = 10
F = 128
Q = 32
ROUNDS = 4
NL = 2 * NV
CPG = NC // NG
VPG = NV // NG
NCORES = 2
NSUB = 16
NW = NCORES * NSUB

_F32 = jnp.float32


def _leaky(x):
    return jnp.where(x >= 0, x, 0.2 * x)


def _full_specs(*arrs):
    return [pl.BlockSpec(a.shape, lambda g, _nd=a.ndim: (0,) * _nd) for a in arrs]


def _mlp_query(literals, noise, w1, b1, w2, b2, w3, b3):
    RQ = 2000

    def body(lp, ln, nz, w1r, b1r, w2r, b2r, w3r, b3r, o, oneg):
        x = jnp.concatenate([lp[...], ln[...], nz[...]], axis=1)
        h = _leaky(x @ w1r[...] + b1r[...])
        h = _leaky(h @ w2r[...] + b2r[...])
        y = h @ w3r[...] + b3r[...]
        o[...] = y
        oneg[...] = -y

    ws = (w1, b1, w2, b2, w3, b3)
    return pl.pallas_call(
        body,
        grid=(NV // RQ,),
        in_specs=[pl.BlockSpec((RQ, F), lambda g: (g, 0)),
                  pl.BlockSpec((RQ, F), lambda g: (NV // RQ + g, 0)),
                  pl.BlockSpec((RQ, 4), lambda g: (g, 0))] + _full_specs(*ws),
        out_specs=[pl.BlockSpec((RQ, 2 * Q), lambda g: (g, 0)),
                   pl.BlockSpec((RQ, 2 * Q), lambda g: (g, 0))],
        out_shape=[jax.ShapeDtypeStruct((NV, 2 * Q), _F32),
                   jax.ShapeDtypeStruct((NV, 2 * Q), _F32)],
    )(literals, literals, noise, *ws)


def _edge_calc(v_e):
    RC = 4200

    def body(v_ref, z_ref, g_ref):
        v = v_ref[...]
        sp = jax.nn.softplus(v)
        cv = (sp[0] + sp[1]) + sp[2]
        z = jnp.exp(-cv)
        z_ref[...] = z
        spg = jnp.where(sp == jnp.inf, 0.0, sp)
        g_ref[...] = (-z)[None] * jnp.exp(v - spg)

    return pl.pallas_call(
        body,
        grid=(NC // RC,),
        in_specs=[pl.BlockSpec((3, RC, 2 * Q), lambda g: (0, g, 0))],
        out_specs=[pl.BlockSpec((RC, 2 * Q), lambda g: (g, 0)),
                   pl.BlockSpec((3, RC, 2 * Q), lambda g: (0, g, 0))],
        out_shape=[jax.ShapeDtypeStruct((NC, 2 * Q), _F32),
                   jax.ShapeDtypeStruct((3, NC, 2 * Q), _F32)],
    )(v_e)


def _mlp_clauses(cstate, closs, w1, b1, w2, b2, w3, b3):
    RC = 4200

    def body(cs, z, w1r, b1r, w2r, b2r, w3r, b3r, o):
        x = jnp.concatenate([cs[...], z[...]], axis=1)
        h = _leaky(x @ w1r[...] + b1r[...])
        h = _leaky(h @ w2r[...] + b2r[...])
        o[...] = h @ w3r[...] + b3r[...]

    ws = (w1, b1, w2, b2, w3, b3)
    return pl.pallas_call(
        body,
        grid=(NC // RC,),
        in_specs=[pl.BlockSpec((RC, F), lambda g: (g, 0)),
                  pl.BlockSpec((RC, 2 * Q), lambda g: (g, 0))] + _full_specs(*ws),
        out_specs=pl.BlockSpec((RC, F + Q), lambda g: (g, 0)),
        out_shape=jax.ShapeDtypeStruct((NC, F + Q), _F32),
    )(cstate, closs, *ws)


def _mlp_literals(literals, lgrad, lloss, w1, b1, w2, b2, w3, b3):
    RL = 2000

    def body(lt, gr, ll, w1r, b1r, w2r, b2r, w3r, b3r, o):
        x = jnp.concatenate([lt[...], gr[...], ll[...]], axis=1)
        h = _leaky(x @ w1r[...] + b1r[...])
        h = _leaky(h @ w2r[...] + b2r[...])
        o[...] = h @ w3r[...] + b3r[...]

    ws = (w1, b1, w2, b2, w3, b3)
    return pl.pallas_call(
        body,
        grid=(NL // RL,),
        in_specs=[pl.BlockSpec((RL, F), lambda g: (g, 0)),
                  pl.BlockSpec((RL, Q), lambda g: (g, 0)),
                  pl.BlockSpec((RL, Q), lambda g: (g, 0))] + _full_specs(*ws),
        out_specs=pl.BlockSpec((RL, F), lambda g: (g, 0)),
        out_shape=jax.ShapeDtypeStruct((NL, F), _F32),
    )(literals, lgrad, lloss, *ws)


def _mlp_output(literals, w1, b1, w2, b2, w3p, b3p):
    RQ = 2000

    def body(lp, ln, w1r, b1r, w2r, b2r, w3r, b3r, o):
        x = jnp.concatenate([lp[...], ln[...]], axis=1)
        h = _leaky(x @ w1r[...] + b1r[...])
        h = _leaky(h @ w2r[...] + b2r[...])
        o[...] = h @ w3r[...] + b3r[...]

    ws = (w1, b1, w2, b2, w3p, b3p)
    return pl.pallas_call(
        body,
        grid=(NV // RQ,),
        in_specs=[pl.BlockSpec((RQ, F), lambda g: (g, 0)),
                  pl.BlockSpec((RQ, F), lambda g: (NV // RQ + g, 0))]
                 + _full_specs(*ws),
        out_specs=pl.BlockSpec((RQ, F), lambda g: (g, 0)),
        out_shape=jax.ShapeDtypeStruct((NV, F), _F32),
    )(literals, literals, *ws)


def _loss_tail(zz):
    def body(z, o):
        zv = z[...]
        pcl = zv * (-jnp.log(1.0 - zv + 1e-8))
        pg = jnp.sum(pcl, axis=1)
        o[0, 0] = jnp.sum(jnp.sqrt(pg + 1e-6)) * (1.0 / ROUNDS)

    return pl.pallas_call(
        body,
        in_specs=[pl.BlockSpec(zz.shape, lambda: (0, 0))],
        out_specs=pl.BlockSpec(memory_space=pltpu.SMEM),
        out_shape=jax.ShapeDtypeStruct((1, 1), _F32),
    )(zz)


@functools.lru_cache(maxsize=1)
def _mesh():
    return plsc.VectorSubcoreMesh(core_axis_name="c", subcore_axis_name="s",
                                  num_cores=NCORES, num_subcores=NSUB)

_CH = 80
_NCH = NC // _CH
_MAXCH = -(-_NCH // NW)


def _edge_gather_sc(q2, lidx_kc):
    @functools.partial(
        pl.kernel,
        out_type=jax.ShapeDtypeStruct((3, NC, 2 * Q), _F32),
        mesh=_mesh(),
        compiler_params=pltpu.CompilerParams(use_tc_tiling_on_sc=False),
        scratch_types=[
            pltpu.VMEM((3, _CH), jnp.int32),
            pltpu.VMEM((3, _CH, 2 * Q), _F32),
            pltpu.SemaphoreType.DMA,
        ],
    )
    def k(q_hbm, lidx_hbm, v_hbm, idx_v, rows_v, sem):
        c = lax.axis_index("c")
        s = lax.axis_index("s")
        wid = c * NSUB + s

        def chunk(i, carry):
            cid = wid + i * NW

            @pl.when(cid < _NCH)
            def _():
                c0 = cid * _CH
                pltpu.sync_copy(lidx_hbm.at[:, pl.ds(c0, _CH)], idx_v)
                d0 = pltpu.async_copy(q_hbm.at[idx_v.at[0]], rows_v.at[0], sem)
                d1 = pltpu.async_copy(q_hbm.at[idx_v.at[1]], rows_v.at[1], sem)
                d2 = pltpu.async_copy(q_hbm.at[idx_v.at[2]], rows_v.at[2], sem)
                d0.wait()
                d1.wait()
                d2.wait()
                pltpu.sync_copy(rows_v.at[0], v_hbm.at[0, pl.ds(c0, _CH)])
                pltpu.sync_copy(rows_v.at[1], v_hbm.at[1, pl.ds(c0, _CH)])
                pltpu.sync_copy(rows_v.at[2], v_hbm.at[2, pl.ds(c0, _CH)])
            return carry

        lax.fori_loop(0, _MAXCH, chunk, 0)

    return k(q2, lidx_kc)


def _grad_scatter_sc(g_e, vidx_kc, zeros_v):
    VSTRIPE = NV // NSUB

    @functools.partial(
        pl.kernel,
        out_type=jax.ShapeDtypeStruct((NCORES, NV, 2 * Q), _F32),
        mesh=_mesh(),
        compiler_params=pltpu.CompilerParams(use_tc_tiling_on_sc=False),
        scratch_types=[
            pltpu.VMEM((3, _CH), jnp.int32),
            pltpu.VMEM((3, _CH, 2 * Q), _F32),
            pltpu.VMEM_SHARED((NV, 2 * Q), _F32),
        ],
    )
    def k(g_hbm, vidx_hbm, z_hbm, vg_hbm, idx_v, rows_v, acc):
        c = lax.axis_index("c")
        s = lax.axis_index("s")
        wid = c * NSUB + s
        pltpu.sync_copy(z_hbm.at[pl.ds(s * VSTRIPE, VSTRIPE)],
                        acc.at[pl.ds(s * VSTRIPE, VSTRIPE)])
        plsc.subcore_barrier()

        def chunk(i, carry):
            cid = wid + i * NW

            @pl.when(cid < _NCH)
            def _():
                c0 = cid * _CH
                pltpu.sync_copy(vidx_hbm.at[:, pl.ds(c0, _CH)], idx_v)
                pltpu.sync_copy(g_hbm.at[:, pl.ds(c0, _CH)], rows_v)
                pltpu.sync_copy(rows_v.at[0], acc.at[idx_v.at[0]], add=True)
                pltpu.sync_copy(rows_v.at[1], acc.at[idx_v.at[1]], add=True)
                pltpu.sync_copy(rows_v.at[2], acc.at[idx_v.at[2]], add=True)
            return carry

        lax.fori_loop(0, _MAXCH, chunk, 0)
        plsc.subcore_barrier()
        pltpu.sync_copy(acc.at[pl.ds(s * VSTRIPE, VSTRIPE)],
                        vg_hbm.at[c, pl.ds(s * VSTRIPE, VSTRIPE)])

    return k(g_e, vidx_kc, zeros_v)


def _lit_scatter_sc(cd, lidx_kc, zeros_l):
    LSTRIPE = NL // NSUB

    @functools.partial(
        pl.kernel,
        out_type=jax.ShapeDtypeStruct((NCORES, NL, Q), _F32),
        mesh=_mesh(),
        compiler_params=pltpu.CompilerParams(use_tc_tiling_on_sc=False),
        scratch_types=[
            pltpu.VMEM((3, _CH), jnp.int32),
            pltpu.VMEM((_CH, Q), _F32),
            pltpu.VMEM_SHARED((NL, Q), _F32),
        ],
    )
    def k(cd_hbm, lidx_hbm, z_hbm, ll_hbm, idx_v, crows_v, acc):
        c = lax.axis_index("c")
        s = lax.axis_index("s")
        wid = c * NSUB + s
        pltpu.sync_copy(z_hbm.at[pl.ds(s * LSTRIPE, LSTRIPE)],
                        acc.at[pl.ds(s * LSTRIPE, LSTRIPE)])
        plsc.subcore_barrier()

        def chunk(i, carry):
            cid = wid + i * NW

            @pl.when(cid < _NCH)
            def _():
                c0 = cid * _CH
                pltpu.sync_copy(cd_hbm.at[pl.ds(c0, _CH)], crows_v)
                pltpu.sync_copy(lidx_hbm.at[:, pl.ds(c0, _CH)], idx_v)
                pltpu.sync_copy(crows_v, acc.at[idx_v.at[0]], add=True)
                pltpu.sync_copy(crows_v, acc.at[idx_v.at[1]], add=True)
                pltpu.sync_copy(crows_v, acc.at[idx_v.at[2]], add=True)
            return carry

        lax.fori_loop(0, _MAXCH, chunk, 0)
        plsc.subcore_barrier()
        pltpu.sync_copy(acc.at[pl.ds(s * LSTRIPE, LSTRIPE)],
                        ll_hbm.at[c, pl.ds(s * LSTRIPE, LSTRIPE)])

    return k(cd, lidx_kc, zeros_l)


_CH3 = 240
_NCH3 = NC // _CH3
_MAX3 = -(-_NCH3 // NW)


def _logit_edge_sc(l2, lidx_kc):
    @functools.partial(
        pl.kernel,
        out_type=jax.ShapeDtypeStruct((NC,), _F32),
        mesh=_mesh(),
        compiler_params=pltpu.CompilerParams(use_tc_tiling_on_sc=False,
                                             needs_layout_passes=False),
        scratch_types=[
            pltpu.VMEM((NL,), _F32),
            pltpu.VMEM((3, _CH3), jnp.int32),
            pltpu.VMEM((_CH3,), _F32),
        ],
    )
    def k(lg_hbm, lidx_hbm, z_hbm, lg_v, idx_v, zout_v):
        c = lax.axis_index("c")
        s = lax.axis_index("s")
        wid = c * NSUB + s
        pltpu.sync_copy(lg_hbm, lg_v)

        def chunk(i, carry):
            cid = wid + i * NW

            @pl.when(cid < _NCH3)
            def _():
                c0 = cid * _CH3
                pltpu.sync_copy(lidx_hbm.at[:, pl.ds(c0, _CH3)], idx_v)

                def vec(t, carry2):
                    sl = pl.ds(t * 16, 16)
                    z = jnp.ones((16,), _F32)
                    for kk in range(3):
                        ei = idx_v[kk, sl]
                        v = plsc.load_gather(lg_v, [ei])
                        z = z * (1.0 / (1.0 + jnp.exp(v)))
                    zout_v[sl] = z
                    return carry2

                lax.fori_loop(0, _CH3 // 16, vec, 0)
                pltpu.sync_copy(zout_v, z_hbm.at[pl.ds(c0, _CH3)])
            return carry

        lax.fori_loop(0, _MAX3, chunk, 0)

    return k(l2, lidx_kc)


def kernel(clauses, variable_count, clauses_count, params):
    flat = clauses.reshape(-1)
    var_idx = jnp.abs(flat) - 1
    sign = jnp.sign(flat).astype(_F32)
    lit_idx = jnp.where(flat > 0, var_idx, NV + var_idx)
    lidx_kc = lit_idx.reshape(NC, 3).T
    vidx_kc = var_idx.reshape(NC, 3).T
    sign_kc = sign.reshape(NC, 3).T
    graph_id = jnp.arange(NG)
    variables_mask = jnp.repeat(graph_id, variable_count, total_repeat_length=NV)
    literals_mask = jnp.concatenate([variables_mask, variables_mask], axis=0)
    clauses_mask = jnp.repeat(graph_id, clauses_count, total_repeat_length=NC)
    var_cnt = variable_count.astype(_F32)
    cls_cnt = clauses_count.astype(_F32)

    def pair_norm(x, mask, counts):
        mean = jax.ops.segment_sum(x, mask, num_segments=NG) / counts[:, None]
        x = x - jnp.take(mean, mask, axis=0)
        variance = jnp.mean(jnp.square(x), axis=-1, keepdims=True)
        return x * jax.lax.rsqrt(variance + 1e-6)

    p = params
    qw1, qw2, qw3 = p['literals_query']
    cw1, cw2, cw3 = p['clauses_update']
    lw1, lw2, lw3 = p['literals_update']
    ow1, ow2, ow3 = p['literals_output']
    o_w3p = jnp.pad(ow3[0], ((0, 0), (0, F - 1)))
    o_b3p = jnp.pad(ow3[1], ((0, F - 1),))

    def row(b):
        return b.reshape(1, -1)

    literals = (jnp.zeros((NL, F), _F32).at[:, 0].set(1.0) - 1.0 / F) \
        * (F ** 0.5) * 0.25
    cstate = (jnp.zeros((NC, F), _F32).at[:, 0].set(1.0) - 1.0 / F) \
        * (F ** 0.5) * 0.25
    zeros_v = jnp.zeros((NV, 2 * Q), _F32)
    zeros_l = jnp.zeros((NL, Q), _F32)
    noise_key = jax.random.key(42)

    zs = []
    for step in range(ROUNDS):
        noise = jax.random.normal(jax.random.fold_in(noise_key, step),
                                  (NV, 4), _F32)
        qpos, qneg = _mlp_query(literals, noise, qw1[0], row(qw1[1]),
                                qw2[0], row(qw2[1]), qw3[0], row(qw3[1]))
        q2 = jnp.concatenate([qpos, qneg], axis=0)
        v_e = _edge_gather_sc(q2, lidx_kc)
        closs, g_u = _edge_calc(v_e)
        g_e = g_u * sign_kc[:, :, None]
        vgp = _grad_scatter_sc(g_e, vidx_kc, zeros_v)
        var_grad = vgp[0] + vgp[1]
        literals_grad = jnp.concatenate([var_grad[:, :Q], var_grad[:, Q:]], axis=0)
        cd = _mlp_clauses(cstate, closs, cw1[0], row(cw1[1]),
                          cw2[0], row(cw2[1]), cw3[0], row(cw3[1]))
        cstate = pair_norm(cd[:, Q:], clauses_mask, cls_cnt) * 0.25 + 0.1 * cstate
        llp = _lit_scatter_sc(cd[:, :Q], lidx_kc, zeros_l)
        literals_loss = llp[0] + llp[1]
        y = _mlp_literals(literals, literals_grad, literals_loss,
                          lw1[0], row(lw1[1]), lw2[0], row(lw2[1]),
                          lw3[0], row(lw3[1]))
        literals = pair_norm(y, literals_mask, 2.0 * var_cnt) * 0.25 \
            + 0.1 * literals
        lg128 = _mlp_output(literals, ow1[0], row(ow1[1]),
                            ow2[0], row(ow2[1]), o_w3p, row(o_b3p))
        lf = lg128[:, 0]
        zs.append(_logit_edge_sc(jnp.concatenate([lf, -lf]), lidx_kc))

    zz = jnp.stack(zs).reshape(ROUNDS * NG, CPG)
    total = _loss_tail(zz)[0, 0]
    return lg128[:, :1], total
